# baseline (speedup 1.0000x reference)
import functools

import jax
import jax.numpy as jnp
from jax import lax
from jax.experimental import pallas as pl
from jax.experimental.pallas import tpu as pltpu
from jax.experimental.pallas import tpu_sc as plsc

NBINS = 5
H = 480
W = 640
B = 8
N = 1_000_000
HWSZ = H * W
G = NBINS * HWSZ

EVBLK = 4_096
KGRID = -(-N // EVBLK)
PROWS = 2 * EVBLK // 128
TROWS = KGRID * PROWS

NSUB = 16
NCORE = 2
BATCH_PER_CORE = B // NCORE
CROWS = 16
NCH = TROWS // CROWS
NTRIPS_LO = NCH // NSUB
NEXTRA = NCH - NTRIPS_LO * NSUB
STRIPE = G // NSUB
ZCHUNK = 6_000


def _tc_points(ev_ref, first_ref, dt_ref, idx_ref, val_ref):
    bprog = pl.program_id(0)
    k = pl.program_id(1)
    e = ev_ref[0]
    t = e[:, 0]
    x = e[:, 1]
    y = e[:, 2]
    p = e[:, 3]
    first = first_ref[bprog]
    dT = dt_ref[bprog]
    ts = (t * jnp.float32(4.0) - first * jnp.float32(4.0)) / dT
    tis = jnp.floor(ts)
    ti = tis.astype(jnp.int32)
    dts = ts - tis
    pol = jnp.where(p == 0.0, jnp.float32(-1.0), p)
    vl = pol * (jnp.float32(1.0) - dts)
    vr = pol * dts
    sidx = x.astype(jnp.int32) + y.astype(jnp.int32) * W
    eid = k * EVBLK + lax.broadcasted_iota(jnp.int32, (EVBLK,), 0)
    inb = eid < N
    ok = inb & (tis >= 0)
    m1 = ok & (tis < NBINS)
    m2 = ok & (tis < NBINS - 1)
    ti1 = jnp.clip(ti, 0, NBINS - 1)
    ti2 = jnp.clip(ti + 1, 0, NBINS - 1)
    i1 = jnp.clip(sidx + ti1 * HWSZ, 0, G - 1)
    i2 = jnp.clip(sidx + ti2 * HWSZ, 0, G - 1)
    v1 = jnp.where(m1, vl, jnp.float32(0.0))
    v2 = jnp.where(m2, vr, jnp.float32(0.0))
    hp = PROWS // 2
    idx_ref[0, :hp, :] = i1.reshape(hp, 128)
    idx_ref[0, hp:, :] = i2.reshape(hp, 128)
    val_ref[0, :hp, :] = v1.reshape(hp, 128)
    val_ref[0, hp:, :] = v2.reshape(hp, 128)


_tc_kernel = pl.pallas_call(
    _tc_points,
    grid=(B, KGRID),
    in_specs=[
        pl.BlockSpec((1, EVBLK, 4), lambda b, k: (b, k, 0)),
        pl.BlockSpec(memory_space=pltpu.SMEM),
        pl.BlockSpec(memory_space=pltpu.SMEM),
    ],
    out_specs=[
        pl.BlockSpec((1, PROWS, 128), lambda b, k: (b, k, 0)),
        pl.BlockSpec((1, PROWS, 128), lambda b, k: (b, k, 0)),
    ],
    out_shape=[
        jax.ShapeDtypeStruct((B, TROWS, 128), jnp.int32),
        jax.ShapeDtypeStruct((B, TROWS, 128), jnp.float32),
    ],
)


def _make_sc_kernel():
    mesh = plsc.VectorSubcoreMesh(core_axis_name="c", subcore_axis_name="s")

    @functools.partial(
        pl.kernel,
        out_type=jax.ShapeDtypeStruct((B, G), jnp.float32),
        mesh=mesh,
        scratch_types=[
            pltpu.VMEM((2 * CROWS, 128), jnp.int32),
            pltpu.VMEM((2 * CROWS, 128), jnp.float32),
            pltpu.VMEM((ZCHUNK,), jnp.float32),
            pltpu.VMEM_SHARED((G,), jnp.float32),
            pltpu.SemaphoreType.DMA,
            pltpu.SemaphoreType.DMA,
        ],
        compiler_params=pltpu.CompilerParams(needs_layout_passes=False),
    )
    def voxel_sc(idx_hbm, val_hbm, out_hbm, idx_v, val_v, zbuf, grid,
                 dma_sem, sc_sem):
        c = lax.axis_index("c")
        s = lax.axis_index("s")

        def zb(i, carry):
            zbuf[pl.ds(i * 16, 16)] = jnp.zeros((16,), jnp.float32)
            return carry
        lax.fori_loop(0, ZCHUNK // 16, zb, 0)

        def batch_body(bi, carry):
            b = c * BATCH_PER_CORE + bi
            def zg(j, cc):
                pltpu.sync_copy(
                    zbuf, grid.at[pl.ds(s * STRIPE + j * ZCHUNK, ZCHUNK)])
                return cc
            lax.fori_loop(0, STRIPE // ZCHUNK, zg, 0)
            plsc.subcore_barrier()

            def copies(k, p):
                row0 = (k * NSUB + s) * CROWS
                ci = pltpu.make_async_copy(
                    idx_hbm.at[b, pl.ds(row0, CROWS)],
                    idx_v.at[pl.ds(p * CROWS, CROWS)], dma_sem)
                cv = pltpu.make_async_copy(
                    val_hbm.at[b, pl.ds(row0, CROWS)],
                    val_v.at[pl.ds(p * CROWS, CROWS)], dma_sem)
                return ci, cv

            def fire(j, ioff):
                pltpu.async_copy(
                    val_v.at[ioff + j], grid.at[idx_v.at[ioff + j]],
                    sc_sem, add=True)

            def drain(j, ioff):
                pltpu.make_async_copy(
                    val_v.at[ioff + j], grid.at[idx_v.at[ioff + j]],
                    sc_sem).wait()

            ntrips = jnp.where(s < NEXTRA, NTRIPS_LO + 1, NTRIPS_LO)
            ci, cv = copies(0, 0)
            ci.start()
            cv.start()

            def do_chunk(k, cc):
                p = k & 1
                ioff = p * CROWS
                ci, cv = copies(k, p)
                ci.wait()
                cv.wait()

                @pl.when(k + 1 < ntrips)
                def _():
                    ni, nv = copies(k + 1, 1 - p)
                    ni.start()
                    nv.start()

                @pl.when(k >= 1)
                def _():
                    def dr(j, c3):
                        drain(j, (1 - p) * CROWS)
                        return c3
                    lax.fori_loop(0, CROWS, dr, 0)

                def fi(j, c3):
                    fire(j, ioff)
                    return c3
                lax.fori_loop(0, CROWS, fi, 0)
                return cc

            lax.fori_loop(0, ntrips, do_chunk, 0)

            last_ioff = ((ntrips - 1) & 1) * CROWS

            def drl(j, cc):
                drain(j, last_ioff)
                return cc
            lax.fori_loop(0, CROWS, drl, 0)

            plsc.subcore_barrier()
            pltpu.sync_copy(grid.at[pl.ds(s * STRIPE, STRIPE)],
                            out_hbm.at[b, pl.ds(s * STRIPE, STRIPE)])
            plsc.subcore_barrier()
            return carry

        lax.fori_loop(0, BATCH_PER_CORE, batch_body, 0)

    return voxel_sc


_voxel_sc = _make_sc_kernel()


@jax.jit
def _run(events_list):
    first = events_list[:, 0, 0]
    last = events_list[:, N - 1, 0]
    d0 = last - first
    dT = jnp.where(d0 == 0.0, jnp.float32(1.0), d0)
    idx_all, val_all = _tc_kernel(events_list, first, dT)
    out = _voxel_sc(idx_all, val_all)
    return out.reshape(B, NBINS, H, W)


def kernel(events_list, device):
    return _run(events_list)


# speedup vs baseline: 9.3706x; 9.3706x over previous
import functools

import jax
import jax.numpy as jnp
from jax import lax
from jax.experimental import pallas as pl
from jax.experimental.pallas import tpu as pltpu
from jax.experimental.pallas import tpu_sc as plsc
from jax.experimental import layout as jlayout

NBINS = 5
H = 480
W = 640
B = 8
N = 1_000_000
HWSZ = H * W
G = NBINS * HWSZ

NSUB = 16
NCORE = 2
BATCH_PER_CORE = B // NCORE
CHUNK = 1600
EVN = CHUNK * 4
NCHUNKS = N // CHUNK
VECS = CHUNK // 16
ROWS = 2 * CHUNK // 128
STRIPE = G // NSUB
ZCHUNK = 6_000


def _make_sc_kernel():
    mesh = plsc.VectorSubcoreMesh(core_axis_name="c", subcore_axis_name="s")

    @functools.partial(
        pl.kernel,
        out_type=jax.ShapeDtypeStruct((B, G), jnp.float32),
        mesh=mesh,
        scratch_types=[
            pltpu.VMEM((2 * EVN,), jnp.float32),
            pltpu.VMEM((2 * ROWS, 128), jnp.int32),
            pltpu.VMEM((2 * ROWS, 128), jnp.float32),
            pltpu.VMEM((ZCHUNK,), jnp.float32),
            pltpu.VMEM((16,), jnp.float32),
            pltpu.VMEM_SHARED((G,), jnp.float32),
            pltpu.SemaphoreType.DMA,
            pltpu.SemaphoreType.DMA,
        ],
        compiler_params=pltpu.CompilerParams(needs_layout_passes=False),
    )
    def voxel_sc(ev_hbm, stamps_hbm, out_hbm, ev_v, idx_v, val_v, zbuf,
                 stamps_v, grid, ev_sem, sc_sem):
        c = lax.axis_index("c")
        s = lax.axis_index("s")

        def zb(i, carry):
            zbuf[pl.ds(i * 16, 16)] = jnp.zeros((16,), jnp.float32)
            return carry
        lax.fori_loop(0, ZCHUNK // 16, zb, 0)
        pltpu.sync_copy(stamps_hbm, stamps_v)

        lanes = lax.broadcasted_iota(jnp.int32, (16,), 0)

        def batch_body(bi, carry):
            b = c * BATCH_PER_CORE + bi
            def zg(j, cc):
                pltpu.sync_copy(
                    zbuf, grid.at[pl.ds(s * STRIPE + j * ZCHUNK, ZCHUNK)])
                return cc
            lax.fori_loop(0, STRIPE // ZCHUNK, zg, 0)
            plsc.subcore_barrier()

            b2 = lanes * 0 + b * 2
            first = plsc.load_gather(stamps_v, [b2])
            last = plsc.load_gather(stamps_v, [b2 + 1])
            d0 = last - first
            dT = jnp.where(d0 == 0.0, jnp.float32(1.0), d0)
            recip = jnp.float32(1.0) / dT
            first4 = first * jnp.float32(4.0)

            ntrips = jnp.where(s == 0, NCHUNKS // NSUB + 1, NCHUNKS // NSUB)

            def ev_copy(k, p):
                start = (k * NSUB + s) * EVN
                return pltpu.make_async_copy(
                    ev_hbm.at[b, pl.ds(start, EVN)],
                    ev_v.at[pl.ds(p * EVN, EVN)], ev_sem)

            def compute_vec(v, eoff, ioff):
                rows4 = eoff + v * 64 + lanes * 4
                t = plsc.load_gather(ev_v, [rows4])
                x = plsc.load_gather(ev_v, [rows4 + 1])
                y = plsc.load_gather(ev_v, [rows4 + 2])
                p = plsc.load_gather(ev_v, [rows4 + 3])
                ts = (t * jnp.float32(4.0) - first4) * recip
                trunc = ts.astype(jnp.int32)
                tf = trunc.astype(jnp.float32)
                ti = jnp.where(ts < tf, trunc - 1, trunc)
                dts = ts - ti.astype(jnp.float32)
                pol = jnp.where(p == 0.0, jnp.float32(-1.0), p)
                vl = pol * (jnp.float32(1.0) - dts)
                vr = pol * dts
                sidx = x.astype(jnp.int32) + y.astype(jnp.int32) * W
                ok = ti >= 0
                m1 = ok & (ti < NBINS)
                m2 = ok & (ti < NBINS - 1)
                ti1 = jnp.clip(ti, 0, NBINS - 1)
                ti2 = jnp.clip(ti + 1, 0, NBINS - 1)
                i1 = sidx + ti1 * HWSZ
                i2 = sidx + ti2 * HWSZ
                v1 = jnp.where(m1, vl, jnp.float32(0.0))
                v2 = jnp.where(m2, vr, jnp.float32(0.0))
                r = ioff + (v >> 2)
                col = (v & 3) * 32
                idx_v[r, pl.ds(col, 16)] = i1
                idx_v[r, pl.ds(col + 16, 16)] = i2
                val_v[r, pl.ds(col, 16)] = v1
                val_v[r, pl.ds(col + 16, 16)] = v2

            def fire(j, ioff):
                pltpu.async_copy(
                    val_v.at[ioff + j], grid.at[idx_v.at[ioff + j]],
                    sc_sem, add=True)

            def drain(j, ioff):
                pltpu.make_async_copy(
                    val_v.at[ioff + j], grid.at[idx_v.at[ioff + j]],
                    sc_sem).wait()

            ev_copy(0, 0).start()

            def do_chunk(k, cc):
                p = k & 1
                eoff = p * EVN
                ioff = p * ROWS
                ev_copy(k, p).wait()

                @pl.when(k + 1 < ntrips)
                def _():
                    ev_copy(k + 1, 1 - p).start()

                def cv(v, c2):
                    compute_vec(v, eoff, ioff)
                    return c2
                lax.fori_loop(0, VECS, cv, 0)

                @pl.when(k >= 1)
                def _():
                    def dr(j, c3):
                        drain(j, (1 - p) * ROWS)
                        return c3
                    lax.fori_loop(0, ROWS, dr, 0)

                def fi(j, c3):
                    fire(j, ioff)
                    return c3
                lax.fori_loop(0, ROWS, fi, 0)
                return cc

            lax.fori_loop(0, ntrips, do_chunk, 0)

            last_ioff = ((ntrips - 1) & 1) * ROWS

            def drl(j, cc):
                drain(j, last_ioff)
                return cc
            lax.fori_loop(0, ROWS, drl, 0)

            plsc.subcore_barrier()
            pltpu.sync_copy(grid.at[pl.ds(s * STRIPE, STRIPE)],
                            out_hbm.at[b, pl.ds(s * STRIPE, STRIPE)])
            plsc.subcore_barrier()
            return carry

        lax.fori_loop(0, BATCH_PER_CORE, batch_body, 0)

    return voxel_sc


_voxel_sc = _make_sc_kernel()


@jax.jit
def _run(events_list):
    stamps = jnp.stack(
        [events_list[:, 0, 0], events_list[:, N - 1, 0]], axis=1)
    ev2 = events_list.reshape(B, N * 4)
    out = _voxel_sc(ev2, stamps.reshape(16))
    return out.reshape(B, NBINS, H, W)


def kernel(events_list, device):
    return _run(events_list)
